# baseline (speedup 1.0000x reference)
import functools

import jax
import jax.numpy as jnp
from jax.experimental import pallas as pl
from jax.experimental.pallas import tpu as pltpu


def _fast_rsqrt(x):
    i = jax.lax.bitcast_convert_type(x, jnp.int32)
    i = jnp.int32(0x5F3759DF) - jax.lax.shift_right_logical(i, 1)
    y = jax.lax.bitcast_convert_type(i, jnp.float32)
    h = 0.5 * x
    y = y * (1.5 - h * y * y)
    y = y * (1.5 - h * y * y)
    y = y * (1.5 - h * y * y)
    return y


def _body(adj_ref, ids_ref, W1_ref, b1_ref, W2_ref, b2_ref, out_ref,
          carry_ref, m_ref, d_ref, ne_ref, eye_ref, *, TS, HF, NC):
    b = pl.program_id(0)
    j = pl.program_id(1)

    Qq = ids_ref.shape[0]
    Dd = ids_ref.shape[-1]

    @pl.when((b == 0) & (j == 0))
    def _():
        k = jax.lax.broadcasted_iota(jnp.int32, (TS, TS), 0)
        s = jax.lax.broadcasted_iota(jnp.int32, (TS, TS), 1)
        dec = jnp.exp2(-(k - s + 1).astype(jnp.float32))
        m_ref[...] = jnp.where(k >= s, dec, 0.0)
        s8 = jax.lax.broadcasted_iota(jnp.int32, (8, TS), 1)
        d_ref[...] = jnp.exp2((s8 - TS).astype(jnp.float32))
        ri = jax.lax.broadcasted_iota(jnp.int32, (Qq, Qq, HF), 0)
        rj = jax.lax.broadcasted_iota(jnp.int32, (Qq, Qq, HF), 1)
        eq = ri == rj
        ne_ref[...] = jnp.where(eq, 0.0, 1.0).reshape(Qq * Qq, HF)
        eye_ref[...] = jnp.where(eq, 1.0, 0.0).reshape(Qq * Qq, HF)

    @pl.when(j == 0)
    def _():
        carry_ref[...] = jnp.zeros_like(carry_ref)

    H1 = jnp.dot(ids_ref[...], W1_ref[...], preferred_element_type=jnp.float32)
    H2 = jnp.dot(H1, W2_ref[...], preferred_element_type=jnp.float32)
    c = jnp.dot(b1_ref[...], W2_ref[...], preferred_element_type=jnp.float32)

    af = adj_ref[0].reshape(Qq * Qq, TS)
    w = (jnp.dot(af, m_ref[...], preferred_element_type=jnp.float32)
         + carry_ref[...] * d_ref[0:1, :])
    carry_ref[...] = jnp.broadcast_to(w[:, 0:1], (Qq * Qq, TS))

    ne = ne_ref[...]
    eye = eye_ref[...]
    for h in range(TS // HF):
        wh = w[:, h * HF:(h + 1) * HF]
        t = wh * ne
        deg = jnp.sum(t.reshape(Qq, Qq, HF), axis=1) + 1.0
        dis = _fast_rsqrt(deg)
        a = t + eye
        an = a.reshape(Qq, Qq, HF) * dis[:, None, :] * dis[None, :, :]

        an_t = jnp.transpose(an, (2, 0, 1))
        anf = an_t.reshape(HF * Qq, Qq)
        y = jnp.dot(anf, H2, preferred_element_type=jnp.float32) + c
        z = jax.lax.dot_general(
            an_t, y.reshape(HF, Qq, Dd),
            dimension_numbers=(((2,), (1,)), ((0,), (0,))),
            preferred_element_type=jnp.float32)
        o = z + b2_ref[...].reshape(1, 1, Dd)
        out_ref[0, :, :, h * HF:(h + 1) * HF] = jnp.transpose(o, (1, 2, 0))


def kernel(adj_matrices, ids, W1, b1, W2, b2):
    B, S, Q, _ = adj_matrices.shape
    D = ids.shape[-1]
    TS = 256
    HF = 128
    NC = S // TS

    adj_t = jnp.transpose(adj_matrices, (0, 2, 3, 1))
    b1r = b1.reshape(1, D)
    b2r = b2.reshape(1, D)

    body = functools.partial(_body, TS=TS, HF=HF, NC=NC)
    out_t = pl.pallas_call(
        body,
        grid=(B, NC),
        in_specs=[
            pl.BlockSpec((1, Q, Q, TS), lambda b, j: (b, 0, 0, NC - 1 - j)),
            pl.BlockSpec((Q, D), lambda b, j: (0, 0)),
            pl.BlockSpec((D, D), lambda b, j: (0, 0)),
            pl.BlockSpec((1, D), lambda b, j: (0, 0)),
            pl.BlockSpec((D, D), lambda b, j: (0, 0)),
            pl.BlockSpec((1, D), lambda b, j: (0, 0)),
        ],
        out_specs=pl.BlockSpec((1, Q, D, TS), lambda b, j: (b, 0, 0, NC - 1 - j)),
        out_shape=jax.ShapeDtypeStruct((B, Q, D, S), jnp.float32),
        scratch_shapes=[
            pltpu.VMEM((Q * Q, TS), jnp.float32),
            pltpu.VMEM((TS, TS), jnp.float32),
            pltpu.VMEM((8, TS), jnp.float32),
            pltpu.VMEM((Q * Q, HF), jnp.float32),
            pltpu.VMEM((Q * Q, HF), jnp.float32),
        ],
        compiler_params=pltpu.CompilerParams(
            dimension_semantics=("parallel", "arbitrary"),
        ),
    )(adj_t, ids, W1, b1r, W2, b2r)
    return jnp.transpose(out_t, (0, 3, 1, 2))


# speedup vs baseline: 1.4666x; 1.4666x over previous
import functools

import jax
import jax.numpy as jnp
from jax.experimental import pallas as pl
from jax.experimental.pallas import tpu as pltpu


def _fast_rsqrt(x):
    i = jax.lax.bitcast_convert_type(x, jnp.int32)
    i = jnp.int32(0x5F3759DF) - jax.lax.shift_right_logical(i, 1)
    y = jax.lax.bitcast_convert_type(i, jnp.float32)
    h = 0.5 * x
    y = y * (1.5 - h * y * y)
    y = y * (1.5 - h * y * y)
    y = y * (1.5 - h * y * y)
    return y


def _body(adj_ref, ids_ref, W1_ref, b1_ref, W2_ref, b2_ref, out_ref,
          carry_ref, m_ref, d_ref, ne_ref, eye_ref, *, TS, HF, NC):
    b = pl.program_id(0)
    j = pl.program_id(1)

    Qq = ids_ref.shape[0]
    Dd = ids_ref.shape[-1]

    @pl.when((b == 0) & (j == 0))
    def _():
        k = jax.lax.broadcasted_iota(jnp.int32, (TS, 2 * TS), 0)
        s = jax.lax.broadcasted_iota(jnp.int32, (TS, 2 * TS), 1)
        sm = jnp.where(s < TS, s, 0)
        dec = jnp.exp2(-(k - sm + 1).astype(jnp.float32))
        m_ref[...] = jnp.where(k >= sm, dec, 0.0)
        s8 = jax.lax.broadcasted_iota(jnp.int32, (8, TS), 1)
        d_ref[...] = jnp.exp2((s8 - TS).astype(jnp.float32))
        ri = jax.lax.broadcasted_iota(jnp.int32, (Qq, Qq, HF), 0)
        rj = jax.lax.broadcasted_iota(jnp.int32, (Qq, Qq, HF), 1)
        eq = ri == rj
        ne_ref[...] = jnp.where(eq, 0.0, 1.0).reshape(Qq * Qq, HF)
        eye_ref[...] = jnp.where(eq, 1.0, 0.0).reshape(Qq * Qq, HF)

    @pl.when(j == 0)
    def _():
        carry_ref[...] = jnp.zeros_like(carry_ref)

    H1 = jnp.dot(ids_ref[...], W1_ref[...], preferred_element_type=jnp.float32)
    H2 = (jnp.dot(H1, W2_ref[...], preferred_element_type=jnp.float32)
          .astype(jnp.bfloat16))
    c = jnp.dot(b1_ref[...], W2_ref[...], preferred_element_type=jnp.float32)

    af = adj_ref[0].reshape(Qq * Qq, TS)
    wx = jnp.dot(af, m_ref[...], preferred_element_type=jnp.float32)
    w = wx[:, :TS] + carry_ref[...] * d_ref[0:1, :]
    carry_ref[...] = wx[:, TS:]

    ne = ne_ref[...]
    eye = eye_ref[...]
    for h in range(TS // HF):
        wh = w[:, h * HF:(h + 1) * HF]
        t = wh * ne
        deg = jnp.sum(t.reshape(Qq, Qq, HF), axis=1) + 1.0
        dis = _fast_rsqrt(deg)
        a = t + eye
        an = (a.reshape(Qq, Qq, HF) * dis[:, None, :] * dis[None, :, :]
              ).astype(jnp.bfloat16)

        an_t = jnp.transpose(an, (2, 0, 1))
        anf = an_t.reshape(HF * Qq, Qq)
        y = jnp.dot(anf, H2, preferred_element_type=jnp.float32) + c
        yb = y.astype(jnp.bfloat16)
        z = jax.lax.dot_general(
            an_t, yb.reshape(HF, Qq, Dd),
            dimension_numbers=(((2,), (1,)), ((0,), (0,))),
            preferred_element_type=jnp.float32)
        o = z + b2_ref[...].reshape(1, 1, Dd)
        ot = jnp.transpose(jnp.transpose(o, (1, 0, 2)), (0, 2, 1))
        out_ref[0, :, :, h * HF:(h + 1) * HF] = ot


def kernel(adj_matrices, ids, W1, b1, W2, b2):
    B, S, Q, _ = adj_matrices.shape
    D = ids.shape[-1]
    TS = 256
    HF = 128
    NC = S // TS

    adj_t = jnp.transpose(adj_matrices, (0, 2, 3, 1))
    b1r = b1.reshape(1, D)
    b2r = b2.reshape(1, D)

    body = functools.partial(_body, TS=TS, HF=HF, NC=NC)
    out_t = pl.pallas_call(
        body,
        grid=(B, NC),
        in_specs=[
            pl.BlockSpec((1, Q, Q, TS), lambda b, j: (b, 0, 0, NC - 1 - j)),
            pl.BlockSpec((Q, D), lambda b, j: (0, 0)),
            pl.BlockSpec((D, D), lambda b, j: (0, 0)),
            pl.BlockSpec((1, D), lambda b, j: (0, 0)),
            pl.BlockSpec((D, D), lambda b, j: (0, 0)),
            pl.BlockSpec((1, D), lambda b, j: (0, 0)),
        ],
        out_specs=pl.BlockSpec((1, Q, D, TS), lambda b, j: (b, 0, 0, NC - 1 - j)),
        out_shape=jax.ShapeDtypeStruct((B, Q, D, S), jnp.float32),
        scratch_shapes=[
            pltpu.VMEM((Q * Q, TS), jnp.float32),
            pltpu.VMEM((TS, 2 * TS), jnp.float32),
            pltpu.VMEM((8, TS), jnp.float32),
            pltpu.VMEM((Q * Q, HF), jnp.float32),
            pltpu.VMEM((Q * Q, HF), jnp.float32),
        ],
        compiler_params=pltpu.CompilerParams(
            dimension_semantics=("parallel", "arbitrary"),
        ),
    )(adj_t, ids, W1, b1r, W2, b2r)
    return jnp.transpose(out_t, (0, 3, 1, 2))


# speedup vs baseline: 1.6989x; 1.1585x over previous
import functools

import jax
import jax.numpy as jnp
from jax.experimental import pallas as pl
from jax.experimental.pallas import tpu as pltpu


def _fast_rsqrt(x):
    i = jax.lax.bitcast_convert_type(x, jnp.int32)
    i = jnp.int32(0x5F3759DF) - jax.lax.shift_right_logical(i, 1)
    y = jax.lax.bitcast_convert_type(i, jnp.float32)
    h = 0.5 * x
    y = y * (1.5 - h * y * y)
    y = y * (1.5 - h * y * y)
    y = y * (1.5 - h * y * y)
    return y


def _body(adj_ref, ids_ref, W1_ref, b1_ref, W2_ref, b2_ref, out_ref,
          carry_ref, m_ref, d_ref, ne_ref, eye_ref, *, TS, HF, NC):
    b = pl.program_id(0)
    j = pl.program_id(1)

    Qq = ids_ref.shape[0]
    Dd = ids_ref.shape[-1]

    @pl.when((b == 0) & (j == 0))
    def _():
        k = jax.lax.broadcasted_iota(jnp.int32, (TS, TS + HF), 0)
        s = jax.lax.broadcasted_iota(jnp.int32, (TS, TS + HF), 1)
        sm = jnp.where(s < TS, s, 0)
        dec = jnp.exp2(-(k - sm + 1).astype(jnp.float32))
        m_ref[...] = jnp.where(k >= sm, dec, 0.0).astype(jnp.bfloat16)
        s8 = jax.lax.broadcasted_iota(jnp.int32, (8, HF), 1)
        d_ref[...] = jnp.exp2((s8 + (TS - HF) - TS).astype(jnp.float32))
        ri = jax.lax.broadcasted_iota(jnp.int32, (Qq, Qq, HF), 0)
        rj = jax.lax.broadcasted_iota(jnp.int32, (Qq, Qq, HF), 1)
        eq = ri == rj
        ne_ref[...] = jnp.where(eq, 0.0, 1.0).reshape(Qq * Qq, HF)
        eye_ref[...] = jnp.where(eq, 1.0, 0.0).reshape(Qq * Qq, HF)

    @pl.when(j == 0)
    def _():
        carry_ref[...] = jnp.zeros_like(carry_ref)

    H1 = jnp.dot(ids_ref[...], W1_ref[...], preferred_element_type=jnp.float32)
    H2 = (jnp.dot(H1, W2_ref[...], preferred_element_type=jnp.float32)
          .astype(jnp.bfloat16))
    c = jnp.dot(b1_ref[...], W2_ref[...], preferred_element_type=jnp.float32)

    af = adj_ref[0].reshape(Qq * Qq, TS).astype(jnp.bfloat16)
    wx = jnp.dot(af, m_ref[...], preferred_element_type=jnp.float32)
    carry_new = wx[:, TS:]
    ne = ne_ref[...]
    eye = eye_ref[...]
    for h in range(TS // HF):
        wh = wx[:, h * HF:(h + 1) * HF]
        if h == TS // HF - 1:
            wh = wh + carry_ref[...] * d_ref[0:1, :]
        t = wh * ne
        deg = jnp.sum(t.reshape(Qq, Qq, HF), axis=1) + 1.0
        dis = _fast_rsqrt(deg)
        a = t + eye
        an = (a.reshape(Qq, Qq, HF) * dis[:, None, :] * dis[None, :, :]
              ).astype(jnp.bfloat16)

        an_t = jnp.transpose(an, (2, 0, 1))
        anf = an_t.reshape(HF * Qq, Qq)
        y = jnp.dot(anf, H2, preferred_element_type=jnp.float32) + c
        yb = y.astype(jnp.bfloat16)
        z = jax.lax.dot_general(
            an_t, yb.reshape(HF, Qq, Dd),
            dimension_numbers=(((2,), (1,)), ((0,), (0,))),
            preferred_element_type=jnp.float32)
        o = (z + b2_ref[...].reshape(1, 1, Dd)).astype(jnp.bfloat16)
        ot = jnp.transpose(jnp.transpose(o, (1, 0, 2)), (0, 2, 1))
        out_ref[0, :, :, h * HF:(h + 1) * HF] = ot.astype(jnp.float32)
    carry_ref[...] = carry_new


def kernel(adj_matrices, ids, W1, b1, W2, b2):
    B, S, Q, _ = adj_matrices.shape
    D = ids.shape[-1]
    TS = 256
    HF = 128
    NC = S // TS

    adj_t = jnp.transpose(adj_matrices, (0, 2, 3, 1))
    b1r = b1.reshape(1, D)
    b2r = b2.reshape(1, D)

    body = functools.partial(_body, TS=TS, HF=HF, NC=NC)
    out_t = pl.pallas_call(
        body,
        grid=(B, NC),
        in_specs=[
            pl.BlockSpec((1, Q, Q, TS), lambda b, j: (b, 0, 0, NC - 1 - j)),
            pl.BlockSpec((Q, D), lambda b, j: (0, 0)),
            pl.BlockSpec((D, D), lambda b, j: (0, 0)),
            pl.BlockSpec((1, D), lambda b, j: (0, 0)),
            pl.BlockSpec((D, D), lambda b, j: (0, 0)),
            pl.BlockSpec((1, D), lambda b, j: (0, 0)),
        ],
        out_specs=pl.BlockSpec((1, Q, D, TS), lambda b, j: (b, 0, 0, NC - 1 - j)),
        out_shape=jax.ShapeDtypeStruct((B, Q, D, S), jnp.float32),
        scratch_shapes=[
            pltpu.VMEM((Q * Q, HF), jnp.float32),
            pltpu.VMEM((TS, TS + HF), jnp.bfloat16),
            pltpu.VMEM((8, HF), jnp.float32),
            pltpu.VMEM((Q * Q, HF), jnp.float32),
            pltpu.VMEM((Q * Q, HF), jnp.float32),
        ],
        compiler_params=pltpu.CompilerParams(
            dimension_semantics=("parallel", "arbitrary"),
        ),
    )(adj_t, ids, W1, b1r, W2, b2r)
    return jnp.transpose(out_t, (0, 3, 1, 2))


# speedup vs baseline: 1.7190x; 1.0118x over previous
import functools

import jax
import jax.numpy as jnp
from jax.experimental import pallas as pl
from jax.experimental.pallas import tpu as pltpu


def _fast_rsqrt(x):
    i = jax.lax.bitcast_convert_type(x, jnp.int32)
    i = jnp.int32(0x5F3759DF) - jax.lax.shift_right_logical(i, 1)
    y = jax.lax.bitcast_convert_type(i, jnp.float32)
    h = 0.5 * x
    y = y * (1.5 - h * y * y)
    y = y * (1.5 - h * y * y)
    return y


def _body(adj_ref, ids_ref, W1_ref, b1_ref, W2_ref, b2_ref, out_ref,
          carry_ref, m_ref, d_ref, eye_ref, *, TS, HF, NC):
    b = pl.program_id(0)
    j = pl.program_id(1)

    Qq = ids_ref.shape[0]
    Dd = ids_ref.shape[-1]

    @pl.when((b == 0) & (j == 0))
    def _():
        k = jax.lax.broadcasted_iota(jnp.int32, (TS, TS + HF), 0)
        s = jax.lax.broadcasted_iota(jnp.int32, (TS, TS + HF), 1)
        sm = jnp.where(s < TS, s, 0)
        dec = jnp.exp2(-(k - sm + 1).astype(jnp.float32))
        m_ref[...] = jnp.where(k >= sm, dec, 0.0).astype(jnp.bfloat16)
        s8 = jax.lax.broadcasted_iota(jnp.int32, (8, HF), 1)
        d_ref[...] = jnp.exp2((s8 + (TS - HF) - TS).astype(jnp.float32))
        ri = jax.lax.broadcasted_iota(jnp.int32, (Qq, Qq, HF), 0)
        rj = jax.lax.broadcasted_iota(jnp.int32, (Qq, Qq, HF), 1)
        eye_ref[...] = (ri == rj).reshape(Qq * Qq, HF)

    @pl.when(j == 0)
    def _():
        carry_ref[...] = jnp.zeros_like(carry_ref)

    H1 = jnp.dot(ids_ref[...], W1_ref[...], preferred_element_type=jnp.float32)
    H2 = (jnp.dot(H1, W2_ref[...], preferred_element_type=jnp.float32)
          .astype(jnp.bfloat16))
    c = jnp.dot(b1_ref[...], W2_ref[...], preferred_element_type=jnp.float32)

    af = adj_ref[0].reshape(Qq * Qq, TS).astype(jnp.bfloat16)
    wx = jnp.dot(af, m_ref[...], preferred_element_type=jnp.float32)
    carry_new = wx[:, TS:]
    eye = eye_ref[...]
    for h in range(TS // HF):
        wh = wx[:, h * HF:(h + 1) * HF]
        if h == TS // HF - 1:
            wh = wh + carry_ref[...] * d_ref[0:1, :]
        a = jnp.where(eye, 1.0, wh)
        deg = jnp.sum(a.reshape(Qq, Qq, HF), axis=1)
        dis = _fast_rsqrt(deg)
        an = (a.reshape(Qq, Qq, HF) * dis[:, None, :] * dis[None, :, :]
              ).astype(jnp.bfloat16)

        an_t = jnp.transpose(an, (2, 0, 1))
        anf = an_t.reshape(HF * Qq, Qq)
        y = jnp.dot(anf, H2, preferred_element_type=jnp.float32) + c
        yb = y.astype(jnp.bfloat16)
        z = jax.lax.dot_general(
            an_t, yb.reshape(HF, Qq, Dd),
            dimension_numbers=(((2,), (1,)), ((0,), (0,))),
            preferred_element_type=jnp.float32)
        o = (z + b2_ref[...].reshape(1, 1, Dd)).astype(jnp.bfloat16)
        ot = jnp.transpose(jnp.transpose(o, (1, 0, 2)), (0, 2, 1))
        out_ref[0, :, :, h * HF:(h + 1) * HF] = ot.astype(jnp.float32)
    carry_ref[...] = carry_new


def kernel(adj_matrices, ids, W1, b1, W2, b2):
    B, S, Q, _ = adj_matrices.shape
    D = ids.shape[-1]
    TS = 256
    HF = 128
    NC = S // TS

    adj_t = jnp.transpose(adj_matrices, (0, 2, 3, 1))
    b1r = b1.reshape(1, D)
    b2r = b2.reshape(1, D)

    body = functools.partial(_body, TS=TS, HF=HF, NC=NC)
    out_t = pl.pallas_call(
        body,
        grid=(B, NC),
        in_specs=[
            pl.BlockSpec((1, Q, Q, TS), lambda b, j: (b, 0, 0, NC - 1 - j)),
            pl.BlockSpec((Q, D), lambda b, j: (0, 0)),
            pl.BlockSpec((D, D), lambda b, j: (0, 0)),
            pl.BlockSpec((1, D), lambda b, j: (0, 0)),
            pl.BlockSpec((D, D), lambda b, j: (0, 0)),
            pl.BlockSpec((1, D), lambda b, j: (0, 0)),
        ],
        out_specs=pl.BlockSpec((1, Q, D, TS), lambda b, j: (b, 0, 0, NC - 1 - j)),
        out_shape=jax.ShapeDtypeStruct((B, Q, D, S), jnp.float32),
        scratch_shapes=[
            pltpu.VMEM((Q * Q, HF), jnp.float32),
            pltpu.VMEM((TS, TS + HF), jnp.bfloat16),
            pltpu.VMEM((8, HF), jnp.float32),
            pltpu.VMEM((Q * Q, HF), jnp.bool_),
        ],
        compiler_params=pltpu.CompilerParams(
            dimension_semantics=("parallel", "arbitrary"),
        ),
    )(adj_t, ids, W1, b1r, W2, b2r)
    return jnp.transpose(out_t, (0, 3, 1, 2))
